# baseline (speedup 1.0000x reference)
import jax
import jax.numpy as jnp
from jax.experimental import pallas as pl
from jax.experimental.pallas import tpu as pltpu

_U_CHUNKS = 32
_I_CHUNKS = 4


def _copy_body(u_in, i_in, u_out, i_out, sem_u, sem_i):
    nu = u_in.shape[0] // _U_CHUNKS
    ni = i_in.shape[0] // _I_CHUNKS
    copies = []
    for k in range(_U_CHUNKS):
        copies.append(pltpu.make_async_copy(
            u_in.at[pl.ds(k * nu, nu)], u_out.at[pl.ds(k * nu, nu)],
            sem_u.at[k]))
    for k in range(_I_CHUNKS):
        copies.append(pltpu.make_async_copy(
            i_in.at[pl.ds(k * ni, ni)], i_out.at[pl.ds(k * ni, ni)],
            sem_i.at[k]))
    for c in copies:
        c.start()
    for c in copies:
        c.wait()


def kernel(perturb, user_emb, item_emb):
    del perturb
    u, i = pl.pallas_call(
        _copy_body,
        in_specs=[
            pl.BlockSpec(memory_space=pl.ANY),
            pl.BlockSpec(memory_space=pl.ANY),
        ],
        out_specs=[
            pl.BlockSpec(memory_space=pl.ANY),
            pl.BlockSpec(memory_space=pl.ANY),
        ],
        out_shape=[
            jax.ShapeDtypeStruct(user_emb.shape, user_emb.dtype),
            jax.ShapeDtypeStruct(item_emb.shape, item_emb.dtype),
        ],
        scratch_shapes=[
            pltpu.SemaphoreType.DMA((_U_CHUNKS,)),
            pltpu.SemaphoreType.DMA((_I_CHUNKS,)),
        ],
    )(user_emb, item_emb)
    return (u, i)


# speedup vs baseline: 16.1468x; 16.1468x over previous
import jax
import jax.numpy as jnp
from jax.experimental import pallas as pl
from jax.experimental.pallas import tpu as pltpu

_GRID = 50


def _copy_body(u_in, i_in, u_out, i_out):
    u_out[...] = u_in[...]
    i_out[...] = i_in[...]


def kernel(perturb, user_emb, item_emb):
    del perturb
    nu = user_emb.shape[0] // _GRID
    ni = item_emb.shape[0] // _GRID
    d = user_emb.shape[1]
    u, i = pl.pallas_call(
        _copy_body,
        grid=(_GRID,),
        in_specs=[
            pl.BlockSpec((nu, d), lambda g: (g, 0)),
            pl.BlockSpec((ni, d), lambda g: (g, 0)),
        ],
        out_specs=[
            pl.BlockSpec((nu, d), lambda g: (g, 0)),
            pl.BlockSpec((ni, d), lambda g: (g, 0)),
        ],
        out_shape=[
            jax.ShapeDtypeStruct(user_emb.shape, user_emb.dtype),
            jax.ShapeDtypeStruct(item_emb.shape, item_emb.dtype),
        ],
    )(user_emb, item_emb)
    return (u, i)


# speedup vs baseline: 16.1607x; 1.0009x over previous
import jax
import jax.numpy as jnp
from jax.experimental import pallas as pl
from jax.experimental.pallas import tpu as pltpu

_BUFS = 8
_BLK = 12500


def _copy_body(u_in, i_in, u_out, i_out, bufs, in_sems, out_sems):
    chunks = []
    for r in range(0, u_in.shape[0], _BLK):
        chunks.append((u_in, u_out, r))
    for r in range(0, i_in.shape[0], _BLK):
        chunks.append((i_in, i_out, r))
    n = len(chunks)

    def in_copy(c, k):
        src, dst, r = chunks[c]
        return pltpu.make_async_copy(
            src.at[pl.ds(r, _BLK)], bufs.at[k], in_sems.at[k])

    def out_copy(c, k):
        src, dst, r = chunks[c]
        return pltpu.make_async_copy(
            bufs.at[k], dst.at[pl.ds(r, _BLK)], out_sems.at[k])

    for k in range(min(_BUFS, n)):
        in_copy(k, k).start()
    for c in range(n):
        k = c % _BUFS
        in_copy(c, k).wait()
        out_copy(c, k).start()
        nxt = c + _BUFS
        if nxt < n:
            out_copy(c, k).wait()
            in_copy(nxt, k).start()
    for c in range(max(0, n - _BUFS), n):
        out_copy(c, c % _BUFS).wait()


def kernel(perturb, user_emb, item_emb):
    del perturb
    u, i = pl.pallas_call(
        _copy_body,
        in_specs=[
            pl.BlockSpec(memory_space=pl.ANY),
            pl.BlockSpec(memory_space=pl.ANY),
        ],
        out_specs=[
            pl.BlockSpec(memory_space=pl.ANY),
            pl.BlockSpec(memory_space=pl.ANY),
        ],
        out_shape=[
            jax.ShapeDtypeStruct(user_emb.shape, user_emb.dtype),
            jax.ShapeDtypeStruct(item_emb.shape, item_emb.dtype),
        ],
        scratch_shapes=[
            pltpu.VMEM((_BUFS, _BLK, 64), jnp.float32),
            pltpu.SemaphoreType.DMA((_BUFS,)),
            pltpu.SemaphoreType.DMA((_BUFS,)),
        ],
    )(user_emb, item_emb)
    return (u, i)
